# Initial kernel scaffold; baseline (speedup 1.0000x reference)
import jax
import jax.numpy as jnp
from jax.experimental import pallas as pl


def kernel(nodes, neigh_idx, node_features, W):
    raise NotImplementedError("write your pallas kernel here")


# speedup vs baseline: 1.5849x; 1.5849x over previous
import functools

import jax
import jax.numpy as jnp
from jax import lax
from jax.experimental import pallas as pl
from jax.experimental.pallas import tpu as pltpu
from jax.experimental.pallas import tpu_sc as plsc

D_FEAT = 128
EMBED = 128
NUM_SAMPLE = 25

NC = 2
NS = 16
NW = NC * NS

PAIRS_PER_OP = 100
BATCH_PER_OP = PAIRS_PER_OP // NUM_SAMPLE
NBUF = 4
COL_CHUNKS = D_FEAT // 16


def _sc_aggregate(table, idx3, b_pad):
  n_ops = idx3.shape[1]
  bpw = b_pad // NW

  mesh = plsc.VectorSubcoreMesh(
      core_axis_name="c", subcore_axis_name="s", num_cores=NC, num_subcores=NS)

  @functools.partial(
      pl.kernel,
      mesh=mesh,
      out_type=jax.ShapeDtypeStruct((b_pad, D_FEAT), jnp.float32),
      scratch_types=[
          pltpu.VMEM((n_ops, PAIRS_PER_OP), jnp.int32),
          pltpu.VMEM((bpw, D_FEAT), jnp.float32),
      ] + [pltpu.VMEM((PAIRS_PER_OP, D_FEAT), jnp.float32) for _ in range(NBUF)]
        + [pltpu.SemaphoreType.DMA for _ in range(NBUF)],
  )
  def agg_kernel(table_hbm, idx_hbm, out_hbm, idx_v, agg_v, *bufs_and_sems):
    bufs = bufs_and_sems[:NBUF]
    sems = bufs_and_sems[NBUF:]
    wid = lax.axis_index("s") * NC + lax.axis_index("c")

    pltpu.sync_copy(idx_hbm.at[wid], idx_v)

    for b in range(NBUF):
      pltpu.async_copy(table_hbm.at[idx_v.at[b]], bufs[b], sems[b])

    def reduce_chunk(j, buf):
      def batch_body(b, _):
        def col_body(c, _):
          cs = c * 16
          acc = buf[b * NUM_SAMPLE, pl.ds(cs, 16)]
          for s in range(1, NUM_SAMPLE):
            acc = acc + buf[b * NUM_SAMPLE + s, pl.ds(cs, 16)]
          agg_v[j * BATCH_PER_OP + b, pl.ds(cs, 16)] = acc
          return 0
        return lax.fori_loop(0, COL_CHUNKS, col_body, 0)
      lax.fori_loop(0, BATCH_PER_OP, batch_body, 0)

    def outer(jo, _):
      for db in range(NBUF):
        j = jo * NBUF + db
        pltpu.make_async_copy(table_hbm.at[idx_v.at[j]], bufs[db], sems[db]).wait()
        reduce_chunk(j, bufs[db])
        nxt = j + NBUF

        @pl.when(nxt < n_ops)
        def _():
          pltpu.async_copy(table_hbm.at[idx_v.at[nxt]], bufs[db], sems[db])
      return 0

    lax.fori_loop(0, n_ops // NBUF, outer, 0)

    pltpu.sync_copy(agg_v, out_hbm.at[pl.ds(wid * bpw, bpw)])

  return agg_kernel(table, idx3)


def _tc_linear_relu(w, agg, b_pad):
  bblk = 2048
  grid = (b_pad // bblk,)

  def mm_body(w_ref, agg_ref, out_ref):
    out_ref[...] = jnp.maximum(
        lax.dot_general(w_ref[...], agg_ref[...],
                        (((1,), (1,)), ((), ())),
                        preferred_element_type=jnp.float32),
        0.0)

  return pl.pallas_call(
      mm_body,
      grid=grid,
      in_specs=[
          pl.BlockSpec((EMBED, D_FEAT), lambda i: (0, 0)),
          pl.BlockSpec((bblk, D_FEAT), lambda i: (i, 0)),
      ],
      out_specs=pl.BlockSpec((EMBED, bblk), lambda i: (0, i)),
      out_shape=jax.ShapeDtypeStruct((EMBED, b_pad), jnp.float32),
  )(w, agg)


def kernel(nodes, neigh_idx, node_features, W):
  batch = neigh_idx.shape[0]
  b_pad = 10240

  idx_flat = neigh_idx.reshape(-1)
  pad = b_pad * NUM_SAMPLE - idx_flat.shape[0]
  idx_flat = jnp.concatenate([idx_flat, jnp.zeros((pad,), jnp.int32)])
  pairs_per_worker = b_pad * NUM_SAMPLE // NW
  idx3 = idx_flat.reshape(NW, pairs_per_worker // PAIRS_PER_OP, PAIRS_PER_OP)

  agg = _sc_aggregate(node_features, idx3, b_pad)
  out = _tc_linear_relu(W * (1.0 / NUM_SAMPLE), agg, b_pad)
  return out[:, :batch]
